# Initial kernel scaffold; baseline (speedup 1.0000x reference)
import functools

import jax
import jax.numpy as jnp
from jax import lax
from jax.experimental import pallas as pl
from jax.experimental.pallas import tpu as pltpu
from jax.experimental.pallas import tpu_sc as plsc

N = 100000
E = 1600000
DX = 49
DE = 16
DH = 4

NC = 2
NS = 16
EPW = E // (NC * NS)
B = 2000
NCHUNK = EPW // B

ROWS_A = 6400
ROWS_B = N - 15 * ROWS_A
ZC = 800

BN = 2000
BE = 8000


def _xw_body(x_ref, w_ref, b_ref, o_ref):
    o_ref[...] = (
        jnp.dot(x_ref[...], w_ref[...], preferred_element_type=jnp.float32)
        + b_ref[...]
    )


def _ew_body(a_ref, w_ref, o_ref):
    o_ref[...] = jnp.dot(a_ref[...], w_ref[...], preferred_element_type=jnp.float32)


def _final_body(x_ref, s_ref, c_ref, o_ref):
    tot = s_ref[0] + s_ref[1]
    cnt = c_ref[0] + c_ref[1]
    mean = tot / jnp.maximum(cnt, 1.0)
    o_ref[...] = jnp.concatenate([x_ref[...], mean], axis=1)


def _sc_body(xw_hbm, ew_hbm, col_hbm, row_hbm, osum, ocnt,
             col_v, row_v, xg_v, ew_v, val_v, ones_v, zb4_v, zb1_v,
             xw_s, ssum_s, scnt_s, sem):
    cid = lax.axis_index("c")
    sid = lax.axis_index("s")

    lanes = lax.iota(jnp.int32, 16)
    zero16 = jnp.zeros((16,), jnp.float32)
    one16 = jnp.ones((16,), jnp.float32)

    q4 = lanes // DH
    r4 = lanes % DH
    def fill_zb4(i, _):
        plsc.store_scatter(zb4_v, [i * (16 // DH) + q4, r4], zero16)
        return 0
    lax.fori_loop(0, (ZC * DH) // 16, fill_zb4, 0)

    def fill_zb1(i, _):
        zb1_v[pl.ds(i * 16, 16)] = zero16
        return 0
    lax.fori_loop(0, ZC // 16, fill_zb1, 0)

    def fill_ones(i, _):
        ones_v[pl.ds(i * 16, 16)] = one16
        return 0
    lax.fori_loop(0, B // 16, fill_ones, 0)

    @pl.when(sid < NS - 1)
    def _():
        off = sid * ROWS_A
        pltpu.sync_copy(xw_hbm.at[pl.ds(off, ROWS_A), :],
                        xw_s.at[pl.ds(off, ROWS_A), :])
        for k in range(ROWS_A // ZC):
            pltpu.sync_copy(zb4_v, ssum_s.at[pl.ds(off + k * ZC, ZC), :])
            pltpu.sync_copy(zb1_v, scnt_s.at[pl.ds(off + k * ZC, ZC)])

    @pl.when(sid == NS - 1)
    def _():
        off = (NS - 1) * ROWS_A
        pltpu.sync_copy(xw_hbm.at[pl.ds(off, ROWS_B), :],
                        xw_s.at[pl.ds(off, ROWS_B), :])
        for k in range(ROWS_B // ZC):
            pltpu.sync_copy(zb4_v, ssum_s.at[pl.ds(off + k * ZC, ZC), :])
            pltpu.sync_copy(zb1_v, scnt_s.at[pl.ds(off + k * ZC, ZC)])

    plsc.subcore_barrier()

    ebase = (cid * NS + sid) * EPW

    def chunk(c, _):
        base = ebase + c * B
        pltpu.sync_copy(col_hbm.at[pl.ds(base, B)], col_v)
        pltpu.sync_copy(row_hbm.at[pl.ds(base, B)], row_v)
        pltpu.sync_copy(ew_hbm.at[pl.ds(base, B), :], ew_v)
        pltpu.async_copy(xw_s.at[col_v], xg_v, sem).wait()

        def comp(i, _):
            i0 = i * (16 // DH) + q4
            a = plsc.load_gather(xg_v, [i0, r4])
            b = plsc.load_gather(ew_v, [i0, r4])
            plsc.store_scatter(val_v, [i0, r4], jnp.maximum(a + b, 0.0))
            return 0
        lax.fori_loop(0, (B * DH) // 16, comp, 0)

        pltpu.sync_copy(val_v, ssum_s.at[row_v], add=True)
        pltpu.sync_copy(ones_v, scnt_s.at[row_v], add=True)
        return 0

    lax.fori_loop(0, NCHUNK, chunk, 0)

    plsc.subcore_barrier()

    @pl.when(sid < NS - 1)
    def _():
        off = sid * ROWS_A
        pltpu.sync_copy(ssum_s.at[pl.ds(off, ROWS_A), :],
                        osum.at[cid, pl.ds(off, ROWS_A), :])
        pltpu.sync_copy(scnt_s.at[pl.ds(off, ROWS_A)],
                        ocnt.at[cid, pl.ds(off, ROWS_A)])

    @pl.when(sid == NS - 1)
    def _():
        off = (NS - 1) * ROWS_A
        pltpu.sync_copy(ssum_s.at[pl.ds(off, ROWS_B), :],
                        osum.at[cid, pl.ds(off, ROWS_B), :])
        pltpu.sync_copy(scnt_s.at[pl.ds(off, ROWS_B)],
                        ocnt.at[cid, pl.ds(off, ROWS_B)])


_sc_call = functools.partial(
    pl.kernel,
    out_type=[
        jax.ShapeDtypeStruct((NC, N, DH), jnp.float32),
        jax.ShapeDtypeStruct((NC, N), jnp.float32),
    ],
    mesh=plsc.VectorSubcoreMesh(core_axis_name="c", subcore_axis_name="s"),
    scratch_types=[
        pltpu.VMEM((B,), jnp.int32),
        pltpu.VMEM((B,), jnp.int32),
        pltpu.VMEM((B, DH), jnp.float32),
        pltpu.VMEM((B, DH), jnp.float32),
        pltpu.VMEM((B, DH), jnp.float32),
        pltpu.VMEM((B,), jnp.float32),
        pltpu.VMEM((ZC, DH), jnp.float32),
        pltpu.VMEM((ZC,), jnp.float32),
        pltpu.VMEM_SHARED((N, DH), jnp.float32),
        pltpu.VMEM_SHARED((N, DH), jnp.float32),
        pltpu.VMEM_SHARED((N,), jnp.float32),
        pltpu.SemaphoreType.DMA,
    ],
)(_sc_body)


@jax.jit
def kernel(x, edge_index, edge_attr, W1, b1):
    w1a = W1[:DX]
    w1b = W1[DX:]
    row = edge_index[0]
    col = edge_index[1]

    xw = pl.pallas_call(
        _xw_body,
        grid=(N // BN,),
        in_specs=[
            pl.BlockSpec((BN, DX), lambda i: (i, 0)),
            pl.BlockSpec((DX, DH), lambda i: (0, 0)),
            pl.BlockSpec((1, DH), lambda i: (0, 0)),
        ],
        out_specs=pl.BlockSpec((BN, DH), lambda i: (i, 0)),
        out_shape=jax.ShapeDtypeStruct((N, DH), jnp.float32),
    )(x, w1a, b1.reshape(1, DH))

    ew = pl.pallas_call(
        _ew_body,
        grid=(E // BE,),
        in_specs=[
            pl.BlockSpec((BE, DE), lambda i: (i, 0)),
            pl.BlockSpec((DE, DH), lambda i: (0, 0)),
        ],
        out_specs=pl.BlockSpec((BE, DH), lambda i: (i, 0)),
        out_shape=jax.ShapeDtypeStruct((E, DH), jnp.float32),
    )(edge_attr, w1b)

    osum, ocnt = _sc_call(xw, ew, col, row)

    out = pl.pallas_call(
        _final_body,
        grid=(N // BN,),
        in_specs=[
            pl.BlockSpec((BN, DX), lambda i: (i, 0)),
            pl.BlockSpec((NC, BN, DH), lambda i: (0, i, 0)),
            pl.BlockSpec((NC, BN, 1), lambda i: (0, i, 0)),
        ],
        out_specs=pl.BlockSpec((BN, DX + DH), lambda i: (i, 0)),
        out_shape=jax.ShapeDtypeStruct((N, DX + DH), jnp.float32),
    )(x, osum, ocnt.reshape(NC, N, 1))

    return out


# speedup vs baseline: 4.5565x; 4.5565x over previous
import jax
import jax.numpy as jnp
from jax import lax
from jax.experimental import pallas as pl
from jax.experimental.pallas import tpu as pltpu
from jax.experimental.pallas import tpu_sc as plsc

N = 100000
E = 1600000
DX = 49
DE = 16
DH = 4

NC = 2
NS = 16
NW = NC * NS

EPW = 49920
BR = 128
RW = (E - NW * EPW) // BR
B = 1920
NCHUNK = EPW // B

NP = 100352
ROWS = NP // NS
ZC = 3584

BN = 2000
BE = 8000


def _xw_body(x_ref, w_ref, b_ref, o_ref):
    o_ref[...] = (
        jnp.dot(x_ref[...], w_ref[...], preferred_element_type=jnp.float32)
        + b_ref[...]
    )


def _ew_body(a_ref, w_ref, o_ref):
    o_ref[...] = jnp.dot(a_ref[...], w_ref[...], preferred_element_type=jnp.float32)


def _final_body(x_ref, s_ref, c_ref, o_ref):
    tot = s_ref[0] + s_ref[1]
    cnt = c_ref[0] + c_ref[1]
    mean = tot / jnp.maximum(cnt, 1.0)
    o_ref[...] = jnp.concatenate([x_ref[...], mean], axis=1)


def _sc_body(xw_hbm, ew_hbm, col_hbm, row_hbm, osum, ocnt,
             col_v, row_v, colx_v, rowx_v, xg_v, ew_v, val_v, ones_v, zb_v,
             xw_s, ssum_s, scnt_s, sem):
    cid = lax.axis_index("c")
    sid = lax.axis_index("s")
    wid = cid * NS + sid

    lanes = lax.iota(jnp.int32, 16)
    zero16 = jnp.zeros((16,), jnp.float32)
    one16 = jnp.ones((16,), jnp.float32)
    q4 = lax.shift_right_logical(lanes, 2)
    r4 = lax.bitwise_and(lanes, 3)

    def fill_zb(i, _):
        zb_v[pl.ds(i * 16, 16)] = zero16
        return 0
    lax.fori_loop(0, ZC // 16, fill_zb, 0)

    def fill_ones(i, _):
        ones_v[pl.ds(i * 16, 16)] = one16
        return 0
    lax.fori_loop(0, B // 16, fill_ones, 0)

    noff = sid * ROWS
    pltpu.sync_copy(xw_hbm.at[pl.ds(noff * DH, ROWS * DH)],
                    xw_s.at[pl.ds(noff * DH, ROWS * DH)])
    for k in range((ROWS * DH) // ZC):
        pltpu.sync_copy(zb_v, ssum_s.at[pl.ds(noff * DH + k * ZC, ZC)])
    pltpu.sync_copy(zb_v, scnt_s.at[pl.ds(noff, ZC)])
    pltpu.sync_copy(zb_v.at[pl.ds(0, ROWS - ZC)],
                    scnt_s.at[pl.ds(noff + ZC, ROWS - ZC)])

    plsc.subcore_barrier()

    def do_chunk(base, nb):
        nbx = nb * DH
        pltpu.sync_copy(col_hbm.at[pl.ds(base, nb)], col_v.at[pl.ds(0, nb)])
        pltpu.sync_copy(row_hbm.at[pl.ds(base, nb)], row_v.at[pl.ds(0, nb)])
        pltpu.sync_copy(ew_hbm.at[pl.ds(base * DH, nbx)],
                        ew_v.at[pl.ds(0, nbx)])

        def build(i, _):
            e16 = i * 4 + q4
            cc = plsc.load_gather(col_v, [e16])
            colx_v[pl.ds(i * 16, 16)] = cc * DH + r4
            rr = plsc.load_gather(row_v, [e16])
            rowx_v[pl.ds(i * 16, 16)] = rr * DH + r4
            return 0
        lax.fori_loop(0, nbx // 16, build, 0)

        pltpu.async_copy(xw_s.at[colx_v.at[pl.ds(0, nbx)]],
                         xg_v.at[pl.ds(0, nbx)], sem).wait()

        def comp(i, _):
            s = pl.ds(i * 16, 16)
            val_v[s] = jnp.maximum(xg_v[s] + ew_v[s], 0.0)
            return 0
        lax.fori_loop(0, nbx // 16, comp, 0)

        pltpu.sync_copy(val_v.at[pl.ds(0, nbx)],
                        ssum_s.at[rowx_v.at[pl.ds(0, nbx)]], add=True)
        pltpu.sync_copy(ones_v.at[pl.ds(0, nb)],
                        scnt_s.at[row_v.at[pl.ds(0, nb)]], add=True)

    ebase = wid * EPW

    def chunk(c, _):
        do_chunk(ebase + c * B, B)
        return 0
    lax.fori_loop(0, NCHUNK, chunk, 0)

    @pl.when(wid < RW)
    def _():
        do_chunk(NW * EPW + wid * BR, BR)

    plsc.subcore_barrier()

    pltpu.sync_copy(ssum_s.at[pl.ds(noff * DH, ROWS * DH)],
                    osum.at[cid, pl.ds(noff * DH, ROWS * DH)])
    pltpu.sync_copy(scnt_s.at[pl.ds(noff, ROWS)],
                    ocnt.at[cid, pl.ds(noff, ROWS)])


_sc_call = pl.kernel(
    _sc_body,
    out_type=[
        jax.ShapeDtypeStruct((NC, NP * DH), jnp.float32),
        jax.ShapeDtypeStruct((NC, NP), jnp.float32),
    ],
    mesh=plsc.VectorSubcoreMesh(core_axis_name="c", subcore_axis_name="s"),
    compiler_params=pltpu.CompilerParams(needs_layout_passes=False),
    scratch_types=[
        pltpu.VMEM((B,), jnp.int32),
        pltpu.VMEM((B,), jnp.int32),
        pltpu.VMEM((B * DH,), jnp.int32),
        pltpu.VMEM((B * DH,), jnp.int32),
        pltpu.VMEM((B * DH,), jnp.float32),
        pltpu.VMEM((B * DH,), jnp.float32),
        pltpu.VMEM((B * DH,), jnp.float32),
        pltpu.VMEM((B,), jnp.float32),
        pltpu.VMEM((ZC,), jnp.float32),
        pltpu.VMEM_SHARED((NP * DH,), jnp.float32),
        pltpu.VMEM_SHARED((NP * DH,), jnp.float32),
        pltpu.VMEM_SHARED((NP,), jnp.float32),
        pltpu.SemaphoreType.DMA,
    ],
)


@jax.jit
def kernel(x, edge_index, edge_attr, W1, b1):
    w1a = W1[:DX]
    w1b = W1[DX:]
    row = edge_index[0]
    col = edge_index[1]

    xw = pl.pallas_call(
        _xw_body,
        grid=(N // BN,),
        in_specs=[
            pl.BlockSpec((BN, DX), lambda i: (i, 0)),
            pl.BlockSpec((DX, DH), lambda i: (0, 0)),
            pl.BlockSpec((1, DH), lambda i: (0, 0)),
        ],
        out_specs=pl.BlockSpec((BN, DH), lambda i: (i, 0)),
        out_shape=jax.ShapeDtypeStruct((N, DH), jnp.float32),
    )(x, w1a, b1.reshape(1, DH))

    ew = pl.pallas_call(
        _ew_body,
        grid=(E // BE,),
        in_specs=[
            pl.BlockSpec((BE, DE), lambda i: (i, 0)),
            pl.BlockSpec((DE, DH), lambda i: (0, 0)),
        ],
        out_specs=pl.BlockSpec((BE, DH), lambda i: (i, 0)),
        out_shape=jax.ShapeDtypeStruct((E, DH), jnp.float32),
    )(edge_attr, w1b)

    xw_p = jnp.pad(xw, ((0, NP - N), (0, 0))).reshape(NP * DH)
    osum, ocnt = _sc_call(xw_p, ew.reshape(E * DH), col, row)

    out = pl.pallas_call(
        _final_body,
        grid=(N // BN,),
        in_specs=[
            pl.BlockSpec((BN, DX), lambda i: (i, 0)),
            pl.BlockSpec((NC, BN, DH), lambda i: (0, i, 0)),
            pl.BlockSpec((NC, BN, 1), lambda i: (0, i, 0)),
        ],
        out_specs=pl.BlockSpec((BN, DX + DH), lambda i: (i, 0)),
        out_shape=jax.ShapeDtypeStruct((N, DX + DH), jnp.float32),
    )(x, osum.reshape(NC, NP, DH), ocnt.reshape(NC, NP, 1))

    return out


# speedup vs baseline: 23.9086x; 5.2471x over previous
import jax
import jax.numpy as jnp
from jax import lax
from jax.experimental import pallas as pl
from jax.experimental.pallas import tpu as pltpu
from jax.experimental.pallas import tpu_sc as plsc

N = 100000
E = 1600000
DX = 49
DE = 16
DH = 4

NC = 2
NS = 16
NW = NC * NS

EPW = 49920
BR = 128
RW = (E - NW * EPW) // BR
B = 3840
NCHUNK = EPW // B

NP = 100352
ROWS = NP // NS
ZC = 3584

BNT = 2048
BET = 16384


def _xw_body(xt_ref, wt_ref, bt_ref, o0, o1, o2, o3):
    r = (
        jnp.dot(wt_ref[...], xt_ref[...], preferred_element_type=jnp.float32)
        + bt_ref[...]
    )
    o0[...] = r[0]
    o1[...] = r[1]
    o2[...] = r[2]
    o3[...] = r[3]


def _ew_body(at_ref, wt_ref, o0, o1, o2, o3):
    r = jnp.dot(wt_ref[...], at_ref[...], preferred_element_type=jnp.float32)
    o0[...] = r[0]
    o1[...] = r[1]
    o2[...] = r[2]
    o3[...] = r[3]


def _final_body(xt_ref, s_ref, c_ref, o_ref):
    tot = s_ref[0] + s_ref[1]
    cnt = c_ref[0] + c_ref[1]
    mean = tot / jnp.maximum(cnt, 1.0)
    o_ref[:DX, :] = xt_ref[...]
    o_ref[DX:, :] = mean


def _sc_body(xw0, xw1, xw2, xw3, ew0, ew1, ew2, ew3, col_hbm, row_hbm,
             osum, ocnt,
             col_v, row_v, xg0, xg1, xg2, xg3, ev0, ev1, ev2, ev3,
             ones_v, zb_v, xs0, xs1, xs2, xs3, ss0, ss1, ss2, ss3,
             scnt_s, sem):
    cid = lax.axis_index("c")
    sid = lax.axis_index("s")
    wid = cid * NS + sid

    xw = [xw0, xw1, xw2, xw3]
    ew = [ew0, ew1, ew2, ew3]
    xg = [xg0, xg1, xg2, xg3]
    ev = [ev0, ev1, ev2, ev3]
    xs = [xs0, xs1, xs2, xs3]
    ss = [ss0, ss1, ss2, ss3]

    zero16 = jnp.zeros((16,), jnp.float32)
    one16 = jnp.ones((16,), jnp.float32)

    def fill_zb(i, _):
        zb_v[pl.ds(i * 16, 16)] = zero16
        return 0
    lax.fori_loop(0, ZC // 16, fill_zb, 0)

    def fill_ones(i, _):
        ones_v[pl.ds(i * 16, 16)] = one16
        return 0
    lax.fori_loop(0, B // 16, fill_ones, 0)

    noff = sid * ROWS
    for d in range(DH):
        pltpu.sync_copy(xw[d].at[pl.ds(noff, ROWS)],
                        xs[d].at[pl.ds(noff, ROWS)])
        pltpu.sync_copy(zb_v, ss[d].at[pl.ds(noff, ZC)])
        pltpu.sync_copy(zb_v.at[pl.ds(0, ROWS - ZC)],
                        ss[d].at[pl.ds(noff + ZC, ROWS - ZC)])
    pltpu.sync_copy(zb_v, scnt_s.at[pl.ds(noff, ZC)])
    pltpu.sync_copy(zb_v.at[pl.ds(0, ROWS - ZC)],
                    scnt_s.at[pl.ds(noff + ZC, ROWS - ZC)])

    plsc.subcore_barrier()

    def do_chunk(base, nb):
        pltpu.sync_copy(col_hbm.at[pl.ds(base, nb)], col_v.at[pl.ds(0, nb)])
        pltpu.sync_copy(row_hbm.at[pl.ds(base, nb)], row_v.at[pl.ds(0, nb)])
        for d in range(DH):
            pltpu.sync_copy(ew[d].at[pl.ds(base, nb)], ev[d].at[pl.ds(0, nb)])

        for d in range(DH):
            pltpu.async_copy(xs[d].at[col_v.at[pl.ds(0, nb)]],
                             xg[d].at[pl.ds(0, nb)], sem).wait()

        def comp(i, _):
            s = pl.ds(i * 16, 16)
            for d in range(DH):
                ev[d][s] = jnp.maximum(xg[d][s] + ev[d][s], 0.0)
            return 0
        lax.fori_loop(0, nb // 16, comp, 0)

        for d in range(DH):
            pltpu.sync_copy(ev[d].at[pl.ds(0, nb)],
                            ss[d].at[row_v.at[pl.ds(0, nb)]], add=True)
        pltpu.sync_copy(ones_v.at[pl.ds(0, nb)],
                        scnt_s.at[row_v.at[pl.ds(0, nb)]], add=True)

    ebase = wid * EPW

    def chunk(c, _):
        do_chunk(ebase + c * B, B)
        return 0
    lax.fori_loop(0, NCHUNK, chunk, 0)

    @pl.when(wid < RW)
    def _():
        do_chunk(NW * EPW + wid * BR, BR)

    plsc.subcore_barrier()

    for d in range(DH):
        pltpu.sync_copy(ss[d].at[pl.ds(noff, ROWS)],
                        osum.at[cid, pl.ds(d * NP + noff, ROWS)])
    pltpu.sync_copy(scnt_s.at[pl.ds(noff, ROWS)],
                    ocnt.at[cid, pl.ds(noff, ROWS)])


_plane = jax.ShapeDtypeStruct((NP,), jnp.float32)
_eplane = jax.ShapeDtypeStruct((E,), jnp.float32)

_sc_call = pl.kernel(
    _sc_body,
    out_type=[
        jax.ShapeDtypeStruct((NC, DH * NP), jnp.float32),
        jax.ShapeDtypeStruct((NC, NP), jnp.float32),
    ],
    mesh=plsc.VectorSubcoreMesh(core_axis_name="c", subcore_axis_name="s"),
    compiler_params=pltpu.CompilerParams(needs_layout_passes=False),
    scratch_types=(
        [pltpu.VMEM((B,), jnp.int32)] * 2
        + [pltpu.VMEM((B,), jnp.float32)] * 4
        + [pltpu.VMEM((B,), jnp.float32)] * 4
        + [
            pltpu.VMEM((B,), jnp.float32),
            pltpu.VMEM((ZC,), jnp.float32),
        ]
        + [pltpu.VMEM_SHARED((NP,), jnp.float32)] * 4
        + [pltpu.VMEM_SHARED((NP,), jnp.float32)] * 4
        + [
            pltpu.VMEM_SHARED((NP,), jnp.float32),
            pltpu.SemaphoreType.DMA,
        ]
    ),
)


@jax.jit
def kernel(x, edge_index, edge_attr, W1, b1):
    xt = x.T
    eat = edge_attr.T
    w1at = W1[:DX].T
    w1bt = W1[DX:].T
    row = edge_index[0]
    col = edge_index[1]

    nblocks = NP // BNT
    xw_planes = pl.pallas_call(
        _xw_body,
        grid=(nblocks,),
        in_specs=[
            pl.BlockSpec((DX, BNT), lambda i: (0, i)),
            pl.BlockSpec((DH, DX), lambda i: (0, 0)),
            pl.BlockSpec((DH, 1), lambda i: (0, 0)),
        ],
        out_specs=[pl.BlockSpec((BNT,), lambda i: (i,))] * DH,
        out_shape=[_plane] * DH,
    )(xt, w1at, b1.reshape(DH, 1))

    ew_planes = pl.pallas_call(
        _ew_body,
        grid=(pl.cdiv(E, BET),),
        in_specs=[
            pl.BlockSpec((DE, BET), lambda i: (0, i)),
            pl.BlockSpec((DH, DE), lambda i: (0, 0)),
        ],
        out_specs=[pl.BlockSpec((BET,), lambda i: (i,))] * DH,
        out_shape=[_eplane] * DH,
    )(eat, w1bt)

    osum, ocnt = _sc_call(*xw_planes, *ew_planes, col, row)
    osum = osum.reshape(NC, DH, NP)
    ocnt = ocnt.reshape(NC, 1, NP)

    out_t = pl.pallas_call(
        _final_body,
        grid=(nblocks,),
        in_specs=[
            pl.BlockSpec((DX, BNT), lambda i: (0, i)),
            pl.BlockSpec((NC, DH, BNT), lambda i: (0, 0, i)),
            pl.BlockSpec((NC, 1, BNT), lambda i: (0, 0, i)),
        ],
        out_specs=pl.BlockSpec((DX + DH, BNT), lambda i: (0, i)),
        out_shape=jax.ShapeDtypeStruct((DX + DH, N), jnp.float32),
    )(xt, osum, ocnt)

    return out_t.T


# speedup vs baseline: 25.1560x; 1.0522x over previous
import jax
import jax.numpy as jnp
from jax import lax
from jax.experimental import pallas as pl
from jax.experimental.pallas import tpu as pltpu
from jax.experimental.pallas import tpu_sc as plsc

N = 100000
E = 1600000
DX = 49
DE = 16
DH = 4

NC = 2
NS = 16
NW = NC * NS

EPW = 49920
BR = 128
RW = (E - NW * EPW) // BR
B = 3840
NCHUNK = EPW // B

NP = 100352
ROWS = NP // NS
ZC = 3584

BNT = 2048
BET = 32768


def _xw_body(xt_ref, wt_ref, bt_ref, o0, o1, o2, o3):
    outs = (o0, o1, o2, o3)
    for d in range(DH):
        r = jnp.dot(wt_ref[d : d + 1, :], xt_ref[...],
                    preferred_element_type=jnp.float32)
        outs[d][...] = (r + bt_ref[d : d + 1, :])[0]


def _ew_body(at_ref, wt_ref, o0, o1, o2, o3):
    outs = (o0, o1, o2, o3)
    for d in range(DH):
        r = jnp.dot(wt_ref[d : d + 1, :], at_ref[...],
                    preferred_element_type=jnp.float32)
        outs[d][...] = r[0]


def _final_body(xt_ref, s0, s1, s2, s3, c_ref, o_ref):
    cnt = jnp.maximum(c_ref[0:1, :] + c_ref[1:2, :], 1.0)
    o_ref[:DX, :] = xt_ref[...]
    for d, s in enumerate((s0, s1, s2, s3)):
        o_ref[DX + d : DX + d + 1, :] = (s[0:1, :] + s[1:2, :]) / cnt


def _sc_body(xw0, xw1, xw2, xw3, ew0, ew1, ew2, ew3, col_hbm, row_hbm,
             os0, os1, os2, os3, ocnt,
             col_v, row_v, xg0, xg1, xg2, xg3, ev0, ev1, ev2, ev3,
             ones_v, zb_v, xs0, xs1, xs2, xs3, ss0, ss1, ss2, ss3,
             scnt_s, sem):
    cid = lax.axis_index("c")
    sid = lax.axis_index("s")
    wid = cid * NS + sid

    xw = [xw0, xw1, xw2, xw3]
    ew = [ew0, ew1, ew2, ew3]
    xg = [xg0, xg1, xg2, xg3]
    ev = [ev0, ev1, ev2, ev3]
    xs = [xs0, xs1, xs2, xs3]
    ss = [ss0, ss1, ss2, ss3]
    osum = [os0, os1, os2, os3]

    zero16 = jnp.zeros((16,), jnp.float32)
    one16 = jnp.ones((16,), jnp.float32)

    def fill_zb(i, _):
        zb_v[pl.ds(i * 16, 16)] = zero16
        return 0
    lax.fori_loop(0, ZC // 16, fill_zb, 0)

    def fill_ones(i, _):
        ones_v[pl.ds(i * 16, 16)] = one16
        return 0
    lax.fori_loop(0, B // 16, fill_ones, 0)

    noff = sid * ROWS
    for d in range(DH):
        pltpu.sync_copy(xw[d].at[pl.ds(noff, ROWS)],
                        xs[d].at[pl.ds(noff, ROWS)])
        pltpu.sync_copy(zb_v, ss[d].at[pl.ds(noff, ZC)])
        pltpu.sync_copy(zb_v.at[pl.ds(0, ROWS - ZC)],
                        ss[d].at[pl.ds(noff + ZC, ROWS - ZC)])
    pltpu.sync_copy(zb_v, scnt_s.at[pl.ds(noff, ZC)])
    pltpu.sync_copy(zb_v.at[pl.ds(0, ROWS - ZC)],
                    scnt_s.at[pl.ds(noff + ZC, ROWS - ZC)])

    plsc.subcore_barrier()

    def do_chunk(base, nb):
        pltpu.sync_copy(col_hbm.at[pl.ds(base, nb)], col_v.at[pl.ds(0, nb)])
        pltpu.sync_copy(row_hbm.at[pl.ds(base, nb)], row_v.at[pl.ds(0, nb)])
        for d in range(DH):
            pltpu.sync_copy(ew[d].at[pl.ds(base, nb)], ev[d].at[pl.ds(0, nb)])

        for d in range(DH):
            pltpu.async_copy(xs[d].at[col_v.at[pl.ds(0, nb)]],
                             xg[d].at[pl.ds(0, nb)], sem).wait()

        def comp(i, _):
            s = pl.ds(i * 16, 16)
            for d in range(DH):
                ev[d][s] = jnp.maximum(xg[d][s] + ev[d][s], 0.0)
            return 0
        lax.fori_loop(0, nb // 16, comp, 0)

        for d in range(DH):
            pltpu.sync_copy(ev[d].at[pl.ds(0, nb)],
                            ss[d].at[row_v.at[pl.ds(0, nb)]], add=True)
        pltpu.sync_copy(ones_v.at[pl.ds(0, nb)],
                        scnt_s.at[row_v.at[pl.ds(0, nb)]], add=True)

    ebase = wid * EPW

    def chunk(c, _):
        do_chunk(ebase + c * B, B)
        return 0
    lax.fori_loop(0, NCHUNK, chunk, 0)

    @pl.when(wid < RW)
    def _():
        do_chunk(NW * EPW + wid * BR, BR)

    plsc.subcore_barrier()

    for d in range(DH):
        pltpu.sync_copy(ss[d].at[pl.ds(noff, ROWS)],
                        osum[d].at[cid, pl.ds(noff, ROWS)])
    pltpu.sync_copy(scnt_s.at[pl.ds(noff, ROWS)],
                    ocnt.at[cid, pl.ds(noff, ROWS)])


_plane = jax.ShapeDtypeStruct((NP,), jnp.float32)
_eplane = jax.ShapeDtypeStruct((E,), jnp.float32)
_oplane = jax.ShapeDtypeStruct((NC, NP), jnp.float32)

_sc_call = pl.kernel(
    _sc_body,
    out_type=[_oplane] * (DH + 1),
    mesh=plsc.VectorSubcoreMesh(core_axis_name="c", subcore_axis_name="s"),
    compiler_params=pltpu.CompilerParams(needs_layout_passes=False),
    scratch_types=(
        [pltpu.VMEM((B,), jnp.int32)] * 2
        + [pltpu.VMEM((B,), jnp.float32)] * 4
        + [pltpu.VMEM((B,), jnp.float32)] * 4
        + [
            pltpu.VMEM((B,), jnp.float32),
            pltpu.VMEM((ZC,), jnp.float32),
        ]
        + [pltpu.VMEM_SHARED((NP,), jnp.float32)] * 4
        + [pltpu.VMEM_SHARED((NP,), jnp.float32)] * 4
        + [
            pltpu.VMEM_SHARED((NP,), jnp.float32),
            pltpu.SemaphoreType.DMA,
        ]
    ),
)


@jax.jit
def kernel(x, edge_index, edge_attr, W1, b1):
    xt = x.T
    eat = edge_attr.T
    w1at = W1[:DX].T
    w1bt = W1[DX:].T
    row = edge_index[0]
    col = edge_index[1]

    nblocks = NP // BNT
    xw_planes = pl.pallas_call(
        _xw_body,
        grid=(nblocks,),
        in_specs=[
            pl.BlockSpec((DX, BNT), lambda i: (0, i)),
            pl.BlockSpec((DH, DX), lambda i: (0, 0)),
            pl.BlockSpec((DH, 1), lambda i: (0, 0)),
        ],
        out_specs=[pl.BlockSpec((BNT,), lambda i: (i,))] * DH,
        out_shape=[_plane] * DH,
    )(xt, w1at, b1.reshape(DH, 1))

    ew_planes = pl.pallas_call(
        _ew_body,
        grid=(pl.cdiv(E, BET),),
        in_specs=[
            pl.BlockSpec((DE, BET), lambda i: (0, i)),
            pl.BlockSpec((DH, DE), lambda i: (0, 0)),
        ],
        out_specs=[pl.BlockSpec((BET,), lambda i: (i,))] * DH,
        out_shape=[_eplane] * DH,
    )(eat, w1bt)

    planes = _sc_call(*xw_planes, *ew_planes, col, row)

    out_t = pl.pallas_call(
        _final_body,
        grid=(nblocks,),
        in_specs=[pl.BlockSpec((DX, BNT), lambda i: (0, i))]
        + [pl.BlockSpec((NC, BNT), lambda i: (0, i))] * (DH + 1),
        out_specs=pl.BlockSpec((DX + DH, BNT), lambda i: (0, i)),
        out_shape=jax.ShapeDtypeStruct((DX + DH, N), jnp.float32),
    )(xt, *planes)

    return out_t.T


# speedup vs baseline: 26.5716x; 1.0563x over previous
import jax
import jax.numpy as jnp
from jax import lax
from jax.experimental import pallas as pl
from jax.experimental.pallas import tpu as pltpu
from jax.experimental.pallas import tpu_sc as plsc

N = 100000
E = 1600000
DX = 49
DE = 16
DH = 4

NC = 2
NS = 16
NW = NC * NS

EPW = 49920
BR = 128
RW = (E - NW * EPW) // BR
B = 1920
NCHUNK = EPW // B

NP = 100352
ROWS = NP // NS
ZC = 3584

BNT = 2048
BET = 32768


def _xw_body(xt_ref, wt_ref, bt_ref, o0, o1, o2, o3):
    outs = (o0, o1, o2, o3)
    for d in range(DH):
        r = jnp.dot(wt_ref[d : d + 1, :], xt_ref[...],
                    preferred_element_type=jnp.float32)
        outs[d][...] = (r + bt_ref[d : d + 1, :])[0]


def _ew_body(at_ref, wt_ref, o0, o1, o2, o3):
    outs = (o0, o1, o2, o3)
    for d in range(DH):
        r = jnp.dot(wt_ref[d : d + 1, :], at_ref[...],
                    preferred_element_type=jnp.float32)
        outs[d][...] = r[0]


def _final_body(xt_ref, s0, s1, s2, s3, c_ref, o_ref):
    cnt = jnp.maximum(c_ref[0:1, :] + c_ref[1:2, :], 1.0)
    o_ref[:DX, :] = xt_ref[...]
    for d, s in enumerate((s0, s1, s2, s3)):
        o_ref[DX + d : DX + d + 1, :] = (s[0:1, :] + s[1:2, :]) / cnt


def _sc_body(xw0, xw1, xw2, xw3, ew0, ew1, ew2, ew3, col_hbm, row_hbm,
             os0, os1, os2, os3, ocnt,
             c0, r0, e00, e01, e02, e03, g00, g01, g02, g03,
             c1, r1, e10, e11, e12, e13, g10, g11, g12, g13,
             c2, r2, e20, e21, e22, e23, g20, g21, g22, g23,
             ones_v, zb_v, xs0, xs1, xs2, xs3, ss0, ss1, ss2, ss3,
             scnt_s, sL0, sL1, sL2, sG0, sG1, sG2, sS0, sS1, sS2):
    cid = lax.axis_index("c")
    sid = lax.axis_index("s")
    wid = cid * NS + sid

    xw = [xw0, xw1, xw2, xw3]
    ew = [ew0, ew1, ew2, ew3]
    xs = [xs0, xs1, xs2, xs3]
    ss = [ss0, ss1, ss2, ss3]
    osum = [os0, os1, os2, os3]
    sets = [
        dict(col=c0, row=r0, ev=[e00, e01, e02, e03], xg=[g00, g01, g02, g03],
             sL=sL0, sG=sG0, sS=sS0),
        dict(col=c1, row=r1, ev=[e10, e11, e12, e13], xg=[g10, g11, g12, g13],
             sL=sL1, sG=sG1, sS=sS1),
        dict(col=c2, row=r2, ev=[e20, e21, e22, e23], xg=[g20, g21, g22, g23],
             sL=sL2, sG=sG2, sS=sS2),
    ]

    zero16 = jnp.zeros((16,), jnp.float32)
    one16 = jnp.ones((16,), jnp.float32)

    def fill_zb(i, _):
        zb_v[pl.ds(i * 16, 16)] = zero16
        return 0
    lax.fori_loop(0, ZC // 16, fill_zb, 0)

    def fill_ones(i, _):
        ones_v[pl.ds(i * 16, 16)] = one16
        return 0
    lax.fori_loop(0, B // 16, fill_ones, 0)

    noff = sid * ROWS
    for d in range(DH):
        pltpu.sync_copy(xw[d].at[pl.ds(noff, ROWS)],
                        xs[d].at[pl.ds(noff, ROWS)])
        pltpu.sync_copy(zb_v, ss[d].at[pl.ds(noff, ZC)])
        pltpu.sync_copy(zb_v.at[pl.ds(0, ROWS - ZC)],
                        ss[d].at[pl.ds(noff + ZC, ROWS - ZC)])
    pltpu.sync_copy(zb_v, scnt_s.at[pl.ds(noff, ZC)])
    pltpu.sync_copy(zb_v.at[pl.ds(0, ROWS - ZC)],
                    scnt_s.at[pl.ds(noff + ZC, ROWS - ZC)])

    plsc.subcore_barrier()

    ebase = wid * EPW

    def fire_loads(c):
        st = sets[c % 3]
        base = ebase + c * B
        ds_ = [
            pltpu.async_copy(col_hbm.at[pl.ds(base, B)], st["col"], st["sL"]),
            pltpu.async_copy(row_hbm.at[pl.ds(base, B)], st["row"], st["sL"]),
        ]
        for d in range(DH):
            ds_.append(pltpu.async_copy(ew[d].at[pl.ds(base, B)],
                                        st["ev"][d], st["sL"]))
        return ds_

    def fire_gathers(c):
        st = sets[c % 3]
        return [
            pltpu.async_copy(xs[d].at[st["col"]], st["xg"][d], st["sG"])
            for d in range(DH)
        ]

    def fire_scatters(c):
        st = sets[c % 3]
        ds_ = [
            pltpu.async_copy(st["ev"][d], ss[d].at[st["row"]], st["sS"],
                             add=True)
            for d in range(DH)
        ]
        ds_.append(pltpu.async_copy(ones_v, scnt_s.at[st["row"]], st["sS"],
                                    add=True))
        return ds_

    pend_ld = {}
    pend_g = {}
    pend_sc = {}

    pend_ld[0] = fire_loads(0)
    for dsc in pend_ld.pop(0):
        dsc.wait()
    pend_g[0] = fire_gathers(0)
    pend_ld[1] = fire_loads(1)

    for c in range(NCHUNK):
        st = sets[c % 3]
        if c + 1 < NCHUNK:
            for dsc in pend_ld.pop(c + 1):
                dsc.wait()
            pend_g[c + 1] = fire_gathers(c + 1)
        for dsc in pend_g.pop(c):
            dsc.wait()

        ev, xg = st["ev"], st["xg"]

        def comp(i, _):
            sl = pl.ds(i * 16, 16)
            for d in range(DH):
                ev[d][sl] = jnp.maximum(xg[d][sl] + ev[d][sl], 0.0)
            return 0
        lax.fori_loop(0, B // 16, comp, 0)

        pend_sc[c] = fire_scatters(c)
        if c >= 1:
            for dsc in pend_sc.pop(c - 1):
                dsc.wait()
            if c + 2 < NCHUNK:
                pend_ld[c + 2] = fire_loads(c + 2)
        elif c + 2 < NCHUNK:
            pend_ld[c + 2] = fire_loads(c + 2)

    for dsc in pend_sc.pop(NCHUNK - 1):
        dsc.wait()

    @pl.when(wid < RW)
    def _():
        base = NW * EPW + wid * BR
        st = sets[0]
        colv, rowv, evl, xg = st["col"], st["row"], st["ev"], st["xg"]
        pltpu.sync_copy(col_hbm.at[pl.ds(base, BR)], colv.at[pl.ds(0, BR)])
        pltpu.sync_copy(row_hbm.at[pl.ds(base, BR)], rowv.at[pl.ds(0, BR)])
        for d in range(DH):
            pltpu.sync_copy(ew[d].at[pl.ds(base, BR)], evl[d].at[pl.ds(0, BR)])
        for d in range(DH):
            pltpu.async_copy(xs[d].at[colv.at[pl.ds(0, BR)]],
                             xg[d].at[pl.ds(0, BR)], st["sG"]).wait()

        def compt(i, _):
            sl = pl.ds(i * 16, 16)
            for d in range(DH):
                evl[d][sl] = jnp.maximum(xg[d][sl] + evl[d][sl], 0.0)
            return 0
        lax.fori_loop(0, BR // 16, compt, 0)
        for d in range(DH):
            pltpu.sync_copy(evl[d].at[pl.ds(0, BR)],
                            ss[d].at[rowv.at[pl.ds(0, BR)]], add=True)
        pltpu.sync_copy(ones_v.at[pl.ds(0, BR)],
                        scnt_s.at[rowv.at[pl.ds(0, BR)]], add=True)

    plsc.subcore_barrier()

    for d in range(DH):
        pltpu.sync_copy(ss[d].at[pl.ds(noff, ROWS)],
                        osum[d].at[cid, pl.ds(noff, ROWS)])
    pltpu.sync_copy(scnt_s.at[pl.ds(noff, ROWS)],
                    ocnt.at[cid, pl.ds(noff, ROWS)])


_plane = jax.ShapeDtypeStruct((NP,), jnp.float32)
_eplane = jax.ShapeDtypeStruct((E,), jnp.float32)
_oplane = jax.ShapeDtypeStruct((NC, NP), jnp.float32)

_sc_call = pl.kernel(
    _sc_body,
    out_type=[_oplane] * (DH + 1),
    mesh=plsc.VectorSubcoreMesh(core_axis_name="c", subcore_axis_name="s"),
    compiler_params=pltpu.CompilerParams(needs_layout_passes=False),
    scratch_types=(
        ([pltpu.VMEM((B,), jnp.int32)] * 2
         + [pltpu.VMEM((B,), jnp.float32)] * 8) * 3
        + [
            pltpu.VMEM((B,), jnp.float32),
            pltpu.VMEM((ZC,), jnp.float32),
        ]
        + [pltpu.VMEM_SHARED((NP,), jnp.float32)] * 4
        + [pltpu.VMEM_SHARED((NP,), jnp.float32)] * 4
        + [pltpu.VMEM_SHARED((NP,), jnp.float32)]
        + [pltpu.SemaphoreType.DMA] * 9
    ),
)


@jax.jit
def kernel(x, edge_index, edge_attr, W1, b1):
    xt = x.T
    eat = edge_attr.T
    w1at = W1[:DX].T
    w1bt = W1[DX:].T
    row = edge_index[0]
    col = edge_index[1]

    nblocks = NP // BNT
    xw_planes = pl.pallas_call(
        _xw_body,
        grid=(nblocks,),
        in_specs=[
            pl.BlockSpec((DX, BNT), lambda i: (0, i)),
            pl.BlockSpec((DH, DX), lambda i: (0, 0)),
            pl.BlockSpec((DH, 1), lambda i: (0, 0)),
        ],
        out_specs=[pl.BlockSpec((BNT,), lambda i: (i,))] * DH,
        out_shape=[_plane] * DH,
    )(xt, w1at, b1.reshape(DH, 1))

    ew_planes = pl.pallas_call(
        _ew_body,
        grid=(pl.cdiv(E, BET),),
        in_specs=[
            pl.BlockSpec((DE, BET), lambda i: (0, i)),
            pl.BlockSpec((DH, DE), lambda i: (0, 0)),
        ],
        out_specs=[pl.BlockSpec((BET,), lambda i: (i,))] * DH,
        out_shape=[_eplane] * DH,
    )(eat, w1bt)

    planes = _sc_call(*xw_planes, *ew_planes, col, row)

    out_t = pl.pallas_call(
        _final_body,
        grid=(nblocks,),
        in_specs=[pl.BlockSpec((DX, BNT), lambda i: (0, i))]
        + [pl.BlockSpec((NC, BNT), lambda i: (0, i))] * (DH + 1),
        out_specs=pl.BlockSpec((DX + DH, BNT), lambda i: (0, i)),
        out_shape=jax.ShapeDtypeStruct((DX + DH, N), jnp.float32),
    )(xt, *planes)

    return out_t.T
